# Initial kernel scaffold; baseline (speedup 1.0000x reference)
import jax
import jax.numpy as jnp
from jax.experimental import pallas as pl


def kernel(data, edge_index, edge_weight, W, b):
    raise NotImplementedError("write your pallas kernel here")


# speedup vs baseline: 16.0997x; 16.0997x over previous
import functools

import jax
import jax.numpy as jnp
from jax import lax
from jax.experimental import pallas as pl
from jax.experimental.pallas import tpu as pltpu
from jax.experimental.pallas import tpu_sc as plsc

N = 10000
E = 320000
D = 128
CO = 128
NC = 2
NS = 16
NW = NC * NS
L = 16

B = 128
NCHUNK = 79
EPT = B * NCHUNK
EPAD = NW * EPT
NP2 = 10112
NPD = 10240
RPT = NP2 // NS
CH = NPD // NS

_mesh = plsc.VectorSubcoreMesh(core_axis_name="c", subcore_axis_name="s")


def _rsqrt16(x):
    i = lax.bitcast_convert_type(x, jnp.int32)
    i = jnp.full((L,), 0x5F3759DF, jnp.int32) - (i >> 1)
    y = lax.bitcast_convert_type(i, jnp.float32)
    for _ in range(3):
        y = y * (1.5 - 0.5 * x * y * y)
    return y


@functools.partial(
    pl.kernel,
    out_type=jax.ShapeDtypeStruct((NC, NPD), jnp.float32),
    mesh=_mesh,
    compiler_params=pltpu.CompilerParams(needs_layout_passes=False),
    scratch_types=[
        pltpu.VMEM((NCHUNK, B), jnp.int32),
        pltpu.VMEM((NCHUNK, B), jnp.float32),
        pltpu.VMEM_SHARED((NPD,), jnp.float32),
    ],
)
def _deg_kernel(colp, ewp, zpd, degpart, idx_v, ew_v, dacc):
    c = lax.axis_index("c")
    s = lax.axis_index("s")
    w = c * NS + s

    @pl.when(s == 0)
    def _():
        pltpu.sync_copy(zpd, dacc)

    pltpu.sync_copy(colp.at[w], idx_v)
    pltpu.sync_copy(ewp.at[w], ew_v)
    plsc.subcore_barrier()

    def body(j, carry):
        pltpu.sync_copy(ew_v.at[j], dacc.at[idx_v.at[j]], add=True)
        return carry

    lax.fori_loop(0, NCHUNK, body, 0)
    plsc.subcore_barrier()
    pltpu.sync_copy(dacc.at[pl.ds(s * CH, CH)],
                    degpart.at[c, pl.ds(s * CH, CH)])


@functools.partial(
    pl.kernel,
    out_type=jax.ShapeDtypeStruct((NW, NCHUNK, B), jnp.float32),
    mesh=_mesh,
    compiler_params=pltpu.CompilerParams(needs_layout_passes=False),
    scratch_types=[
        pltpu.VMEM((NCHUNK, B), jnp.int32),
        pltpu.VMEM((NCHUNK, B), jnp.int32),
        pltpu.VMEM((NCHUNK, B), jnp.float32),
        pltpu.VMEM((NCHUNK, B), jnp.float32),
        pltpu.VMEM((CH,), jnp.float32),
        pltpu.VMEM((CH,), jnp.float32),
        pltpu.VMEM((NPD,), jnp.float32),
        pltpu.VMEM_SHARED((NPD,), jnp.float32),
    ],
)
def _wn_kernel(degpart, rowp, colp, ewp, wn_out,
               idxr, idxc, ew_v, wn_v, d0, d1, dis_v, dis_sh):
    c = lax.axis_index("c")
    s = lax.axis_index("s")
    w = c * NS + s
    pltpu.sync_copy(degpart.at[0, pl.ds(s * CH, CH)], d0)
    pltpu.sync_copy(degpart.at[1, pl.ds(s * CH, CH)], d1)
    for i in range(CH // L):
        sl = pl.ds(i * L, L)
        deg = 1.0 + d0[sl] + d1[sl]
        d0[sl] = _rsqrt16(deg)
    pltpu.sync_copy(d0, dis_sh.at[pl.ds(s * CH, CH)])
    plsc.subcore_barrier()
    pltpu.sync_copy(dis_sh, dis_v)
    pltpu.sync_copy(rowp.at[w], idxr)
    pltpu.sync_copy(colp.at[w], idxc)
    pltpu.sync_copy(ewp.at[w], ew_v)

    def body(j, carry):
        for k in range(B // L):
            sl = pl.ds(k * L, L)
            dr = plsc.load_gather(dis_v, [idxr[j, sl]])
            dc = plsc.load_gather(dis_v, [idxc[j, sl]])
            wn_v[j, sl] = dr * ew_v[j, sl] * dc
        return carry

    lax.fori_loop(0, NCHUNK, body, 0)
    pltpu.sync_copy(wn_v, wn_out.at[w])


@functools.partial(
    pl.kernel,
    out_type=jax.ShapeDtypeStruct((NC, NP2, D), jnp.float32),
    mesh=_mesh,
    compiler_params=pltpu.CompilerParams(needs_layout_passes=False),
    scratch_types=[
        pltpu.VMEM((NCHUNK, B), jnp.int32),
        pltpu.VMEM((NCHUNK, B), jnp.int32),
        pltpu.VMEM((NCHUNK, B), jnp.float32),
        pltpu.VMEM((B, D), jnp.float32),
        pltpu.VMEM_SHARED((NP2, D), jnp.float32),
        pltpu.SemaphoreType.DMA,
    ],
)
def _spmm_kernel(xp, rowp, colp, wnp, znd, part,
                 idxr, idxc, wn_v, rows_v, acc, sem):
    c = lax.axis_index("c")
    s = lax.axis_index("s")
    w = c * NS + s
    pltpu.sync_copy(znd.at[pl.ds(s * RPT, RPT)], acc.at[pl.ds(s * RPT, RPT)])
    pltpu.sync_copy(rowp.at[w], idxr)
    pltpu.sync_copy(colp.at[w], idxc)
    pltpu.sync_copy(wnp.at[w], wn_v)
    plsc.subcore_barrier()

    def chunk(j, carry):
        pltpu.async_copy(xp.at[idxc.at[j]], rows_v, sem).wait()

        def escale(g, icarry):
            wv = wn_v[j, pl.ds(g * L, L)]
            base = g * L
            for i in range(L):
                tv = jnp.broadcast_to(wv[i], (L,))
                for k in range(D // L):
                    sl = pl.ds(k * L, L)
                    rows_v[base + i, sl] = rows_v[base + i, sl] * tv
            return icarry

        lax.fori_loop(0, B // L, escale, 0)
        pltpu.sync_copy(rows_v, acc.at[idxr.at[j]], add=True)
        return carry

    lax.fori_loop(0, NCHUNK, chunk, 0)
    plsc.subcore_barrier()
    pltpu.sync_copy(acc.at[pl.ds(s * RPT, RPT)],
                    part.at[c, pl.ds(s * RPT, RPT)])


def _diag_scale(dp, x):
    iv = 1.0 / (1.0 + dp[0, 0, :] + dp[0, 1, :])
    rr = lax.broadcasted_iota(jnp.int32, (128, 128), 0)
    cc = lax.broadcasted_iota(jnp.int32, (128, 128), 1)
    dg = jnp.where(rr == cc, jnp.broadcast_to(iv[None, :], (128, 128)), 0.0)
    return jnp.dot(dg, x, preferred_element_type=jnp.float32)


def _combine_body(dp_ref, p_ref, x_ref, o_ref):
    o_ref[...] = (p_ref[0] + p_ref[1]
                  + _diag_scale(dp_ref[...], x_ref[...]))


_combine_call = pl.pallas_call(
    _combine_body,
    grid=(NP2 // 128,),
    in_specs=[
        pl.BlockSpec((1, NC, 128), lambda j: (j, 0, 0)),
        pl.BlockSpec((NC, 128, D), lambda j: (0, j, 0)),
        pl.BlockSpec((128, D), lambda j: (j, 0)),
    ],
    out_specs=pl.BlockSpec((128, D), lambda j: (j, 0)),
    out_shape=jax.ShapeDtypeStruct((NP2, D), jnp.float32),
)


def _final_body(dp_ref, q_ref, x1_ref, w_ref, b_ref, o_ref):
    h = q_ref[0] + q_ref[1] + _diag_scale(dp_ref[...], x1_ref[...])
    o_ref[...] = lax.dot_general(
        h, w_ref[...], (((1,), (1,)), ((), ())),
        preferred_element_type=jnp.float32) + b_ref[...]


_final_call = pl.pallas_call(
    _final_body,
    grid=(NP2 // 128,),
    in_specs=[
        pl.BlockSpec((1, NC, 128), lambda j: (j, 0, 0)),
        pl.BlockSpec((NC, 128, D), lambda j: (0, j, 0)),
        pl.BlockSpec((128, D), lambda j: (j, 0)),
        pl.BlockSpec((CO, D), lambda j: (0, 0)),
        pl.BlockSpec((1, CO), lambda j: (0, 0)),
    ],
    out_specs=pl.BlockSpec((128, CO), lambda j: (j, 0)),
    out_shape=jax.ShapeDtypeStruct((NP2, CO), jnp.float32),
)


def kernel(data, edge_index, edge_weight, W, b):
    row = edge_index[0]
    col = edge_index[1]
    npad = EPAD - E
    pidx = jnp.arange(npad, dtype=jnp.int32)
    rowp = jnp.concatenate([row, pidx]).reshape(NW, NCHUNK, B)
    colp = jnp.concatenate([col, pidx]).reshape(NW, NCHUNK, B)
    ewp = jnp.concatenate(
        [edge_weight, jnp.zeros((npad,), jnp.float32)]).reshape(NW, NCHUNK, B)
    xp = jnp.concatenate(
        [data, jnp.zeros((NP2 - N, D), jnp.float32)], axis=0)
    zpd = jnp.zeros((NPD,), jnp.float32)
    znd = jnp.zeros((NP2, D), jnp.float32)

    degpart = _deg_kernel(colp, ewp, zpd)
    wn = _wn_kernel(degpart, rowp, colp, ewp)
    dpr = degpart.reshape(NC, NPD // 128, 128).transpose(1, 0, 2)

    part1 = _spmm_kernel(xp, rowp, colp, wn, znd)
    x1 = _combine_call(dpr, part1, xp)
    part2 = _spmm_kernel(x1, rowp, colp, wn, znd)
    out = _final_call(dpr, part2, x1, W, b.reshape(1, CO))
    return out[:N]


# speedup vs baseline: 22.3788x; 1.3900x over previous
import functools

import jax
import jax.numpy as jnp
from jax import lax
from jax.experimental import pallas as pl
from jax.experimental.pallas import tpu as pltpu
from jax.experimental.pallas import tpu_sc as plsc

N = 10000
E = 320000
D = 128
CO = 128
NC = 2
NS = 16
NW = NC * NS
L = 16

B = 128
NCHUNK = 80
EPT = B * NCHUNK
EPAD = NW * EPT
NP2 = 10112
NPD = 10240
RPT = NP2 // NS
CH = NPD // NS

_mesh = plsc.VectorSubcoreMesh(core_axis_name="c", subcore_axis_name="s")


def _rsqrt16(x):
    i = lax.bitcast_convert_type(x, jnp.int32)
    i = jnp.full((L,), 0x5F3759DF, jnp.int32) - (i >> 1)
    y = lax.bitcast_convert_type(i, jnp.float32)
    for _ in range(3):
        y = y * (1.5 - 0.5 * x * y * y)
    return y


@functools.partial(
    pl.kernel,
    out_type=jax.ShapeDtypeStruct((NC, NPD), jnp.float32),
    mesh=_mesh,
    compiler_params=pltpu.CompilerParams(needs_layout_passes=False),
    scratch_types=[
        pltpu.VMEM((NCHUNK, B), jnp.int32),
        pltpu.VMEM((NCHUNK, B), jnp.float32),
        pltpu.VMEM_SHARED((NPD,), jnp.float32),
    ],
)
def _deg_kernel(colp, ewp, zpd, degpart, idx_v, ew_v, dacc):
    c = lax.axis_index("c")
    s = lax.axis_index("s")
    w = c * NS + s

    @pl.when(s == 0)
    def _():
        pltpu.sync_copy(zpd, dacc)

    pltpu.sync_copy(colp.at[w], idx_v)
    pltpu.sync_copy(ewp.at[w], ew_v)
    plsc.subcore_barrier()

    def body(j, carry):
        pltpu.sync_copy(ew_v.at[j], dacc.at[idx_v.at[j]], add=True)
        return carry

    lax.fori_loop(0, NCHUNK, body, 0)
    plsc.subcore_barrier()
    pltpu.sync_copy(dacc.at[pl.ds(s * CH, CH)],
                    degpart.at[c, pl.ds(s * CH, CH)])


@functools.partial(
    pl.kernel,
    out_type=jax.ShapeDtypeStruct((NW, NCHUNK, B), jnp.float32),
    mesh=_mesh,
    compiler_params=pltpu.CompilerParams(needs_layout_passes=False),
    scratch_types=[
        pltpu.VMEM((NCHUNK, B), jnp.int32),
        pltpu.VMEM((NCHUNK, B), jnp.int32),
        pltpu.VMEM((NCHUNK, B), jnp.float32),
        pltpu.VMEM((NCHUNK, B), jnp.float32),
        pltpu.VMEM((CH,), jnp.float32),
        pltpu.VMEM((CH,), jnp.float32),
        pltpu.VMEM((NPD,), jnp.float32),
        pltpu.VMEM_SHARED((NPD,), jnp.float32),
    ],
)
def _wn_kernel(degpart, rowp, colp, ewp, wn_out,
               idxr, idxc, ew_v, wn_v, d0, d1, dis_v, dis_sh):
    c = lax.axis_index("c")
    s = lax.axis_index("s")
    w = c * NS + s
    pltpu.sync_copy(degpart.at[0, pl.ds(s * CH, CH)], d0)
    pltpu.sync_copy(degpart.at[1, pl.ds(s * CH, CH)], d1)
    for i in range(CH // L):
        sl = pl.ds(i * L, L)
        deg = 1.0 + d0[sl] + d1[sl]
        d0[sl] = _rsqrt16(deg)
    pltpu.sync_copy(d0, dis_sh.at[pl.ds(s * CH, CH)])
    plsc.subcore_barrier()
    pltpu.sync_copy(dis_sh, dis_v)
    pltpu.sync_copy(rowp.at[w], idxr)
    pltpu.sync_copy(colp.at[w], idxc)
    pltpu.sync_copy(ewp.at[w], ew_v)

    def body(j, carry):
        for k in range(B // L):
            sl = pl.ds(k * L, L)
            dr = plsc.load_gather(dis_v, [idxr[j, sl]])
            dc = plsc.load_gather(dis_v, [idxc[j, sl]])
            wn_v[j, sl] = dr * ew_v[j, sl] * dc
        return carry

    lax.fori_loop(0, NCHUNK, body, 0)
    pltpu.sync_copy(wn_v, wn_out.at[w])


@functools.partial(
    pl.kernel,
    out_type=jax.ShapeDtypeStruct((NC, NP2, D), jnp.float32),
    mesh=_mesh,
    compiler_params=pltpu.CompilerParams(needs_layout_passes=False),
    scratch_types=[
        pltpu.VMEM((9, B), jnp.int32),
        pltpu.VMEM((B, D), jnp.float32),
        pltpu.VMEM((B, D), jnp.float32),
        pltpu.VMEM_SHARED((NP2, D), jnp.float32),
        pltpu.SemaphoreType.DMA,
        pltpu.SemaphoreType.DMA,
        pltpu.SemaphoreType.DMA,
        pltpu.SemaphoreType.DMA,
        pltpu.SemaphoreType.DMA,
        pltpu.SemaphoreType.DMA,
        pltpu.SemaphoreType.DMA,
    ],
)
def _spmm_kernel(xp, meta, znd, part,
                 meta_v, rows0, rows1, acc,
                 sm0, sm1, sm2, sg0, sg1, ss0, ss1):
    c = lax.axis_index("c")
    s = lax.axis_index("s")
    w = c * NS + s
    rows = (rows0, rows1)
    sm = (sm0, sm1, sm2)
    sg = (sg0, sg1)
    ss = (ss0, ss1)
    pltpu.sync_copy(znd.at[pl.ds(s * RPT, RPT)], acc.at[pl.ds(s * RPT, RPT)])
    plsc.subcore_barrier()

    def meta_load(j, m):
        pltpu.async_copy(meta.at[w, j], meta_v.at[pl.ds(3 * m, 3)], sm[m])

    def wait_meta(j, m):
        pltpu.make_async_copy(meta.at[w, j], meta_v.at[pl.ds(3 * m, 3)],
                              sm[m]).wait()

    def gather(j, m, p):
        pltpu.async_copy(xp.at[meta_v.at[3 * m]], rows[p], sg[p])

    def wait_gather(p):
        pltpu.make_async_copy(xp.at[meta_v.at[0]], rows[p], sg[p]).wait()

    def scatter(m, p):
        pltpu.async_copy(rows[p], acc.at[meta_v.at[3 * m + 1]], ss[p],
                         add=True)

    def wait_scatter(p):
        pltpu.make_async_copy(rows[p], acc.at[pl.ds(0, B)], ss[p]).wait()

    def scale(m, p):
        rows_v = rows[p]

        def escale(g, icarry):
            wv = lax.bitcast_convert_type(
                meta_v[3 * m + 2, pl.ds(g * L, L)], jnp.float32)
            base = g * L
            for i in range(L):
                tv = jnp.broadcast_to(wv[i], (L,))
                for k in range(D // L):
                    sl = pl.ds(k * L, L)
                    rows_v[base + i, sl] = rows_v[base + i, sl] * tv
            return icarry

        lax.fori_loop(0, B // L, escale, 0)

    meta_load(0, 0)
    meta_load(1, 1)
    wait_meta(0, 0)
    gather(0, 0, 0)
    wait_meta(1, 1)
    gather(1, 1, 1)
    wait_gather(0)
    scale(0, 0)
    scatter(0, 0)
    meta_load(2, 2)

    def step(j, pm1, m, m2, p):
        wait_meta(j + 1, pm1)
        wait_scatter(1 - p)
        gather(j + 1, pm1, 1 - p)
        wait_gather(p)
        scale(m, p)
        scatter(m, p)
        meta_load(j + 2, m2)

    def six(j6, carry):
        jb = 1 + 6 * j6
        for u in range(6):
            step(jb + u, (2 + u) % 3, (1 + u) % 3, u % 3, (1 + u) % 2)
        return carry

    lax.fori_loop(0, (NCHUNK - 2) // 6, six, 0)
    wait_meta(NCHUNK, (NCHUNK) % 3)
    wait_scatter(0)
    wait_gather(1)
    scale((NCHUNK - 1) % 3, 1)
    scatter((NCHUNK - 1) % 3, 1)
    wait_scatter(1)
    plsc.subcore_barrier()
    pltpu.sync_copy(acc.at[pl.ds(s * RPT, RPT)],
                    part.at[c, pl.ds(s * RPT, RPT)])


def _diag_scale(dp, x):
    iv = 1.0 / (1.0 + dp[0, 0, :] + dp[0, 1, :])
    rr = lax.broadcasted_iota(jnp.int32, (128, 128), 0)
    cc = lax.broadcasted_iota(jnp.int32, (128, 128), 1)
    dg = jnp.where(rr == cc, jnp.broadcast_to(iv[None, :], (128, 128)), 0.0)
    return jnp.dot(dg, x, preferred_element_type=jnp.float32)


def _combine_body(dp_ref, p_ref, x_ref, o_ref):
    o_ref[...] = (p_ref[0] + p_ref[1]
                  + _diag_scale(dp_ref[...], x_ref[...]))


_combine_call = pl.pallas_call(
    _combine_body,
    grid=(NP2 // 128,),
    in_specs=[
        pl.BlockSpec((1, NC, 128), lambda j: (j, 0, 0)),
        pl.BlockSpec((NC, 128, D), lambda j: (0, j, 0)),
        pl.BlockSpec((128, D), lambda j: (j, 0)),
    ],
    out_specs=pl.BlockSpec((128, D), lambda j: (j, 0)),
    out_shape=jax.ShapeDtypeStruct((NP2, D), jnp.float32),
)


def _final_body(dp_ref, q_ref, x1_ref, w_ref, b_ref, o_ref):
    h = q_ref[0] + q_ref[1] + _diag_scale(dp_ref[...], x1_ref[...])
    o_ref[...] = lax.dot_general(
        h, w_ref[...], (((1,), (1,)), ((), ())),
        preferred_element_type=jnp.float32) + b_ref[...]


_final_call = pl.pallas_call(
    _final_body,
    grid=(NP2 // 128,),
    in_specs=[
        pl.BlockSpec((1, NC, 128), lambda j: (j, 0, 0)),
        pl.BlockSpec((NC, 128, D), lambda j: (0, j, 0)),
        pl.BlockSpec((128, D), lambda j: (j, 0)),
        pl.BlockSpec((CO, D), lambda j: (0, 0)),
        pl.BlockSpec((1, CO), lambda j: (0, 0)),
    ],
    out_specs=pl.BlockSpec((128, CO), lambda j: (j, 0)),
    out_shape=jax.ShapeDtypeStruct((NP2, CO), jnp.float32),
)


def kernel(data, edge_index, edge_weight, W, b):
    row = edge_index[0]
    col = edge_index[1]
    npad = EPAD - E
    pidx = jnp.arange(npad, dtype=jnp.int32)
    rowp = jnp.concatenate([row, pidx]).reshape(NW, NCHUNK, B)
    colp = jnp.concatenate([col, pidx]).reshape(NW, NCHUNK, B)
    ewp = jnp.concatenate(
        [edge_weight, jnp.zeros((npad,), jnp.float32)]).reshape(NW, NCHUNK, B)
    xp = jnp.concatenate(
        [data, jnp.zeros((NP2 - N, D), jnp.float32)], axis=0)
    zpd = jnp.zeros((NPD,), jnp.float32)
    znd = jnp.zeros((NP2, D), jnp.float32)

    degpart = _deg_kernel(colp, ewp, zpd)
    wn = _wn_kernel(degpart, rowp, colp, ewp)
    dpr = degpart.reshape(NC, NPD // 128, 128).transpose(1, 0, 2)

    wni = lax.bitcast_convert_type(wn, jnp.int32)
    meta = jnp.stack([colp, rowp, wni], axis=2)
    meta = jnp.concatenate(
        [meta, jnp.zeros((NW, 1, 3, B), jnp.int32)], axis=1)

    part1 = _spmm_kernel(xp, meta, znd)
    x1 = _combine_call(dpr, part1, xp)
    part2 = _spmm_kernel(x1, meta, znd)
    out = _final_call(dpr, part2, x1, W, b.reshape(1, CO))
    return out[:N]


# speedup vs baseline: 26.9913x; 1.2061x over previous
import functools

import jax
import jax.numpy as jnp
from jax import lax
from jax.experimental import pallas as pl
from jax.experimental.pallas import tpu as pltpu
from jax.experimental.pallas import tpu_sc as plsc

N = 10000
E = 320000
D = 128
CO = 128
NC = 2
NS = 16
NW = NC * NS
L = 16

B = 128
NCHUNK = 80
EPT = B * NCHUNK
EPAD = NW * EPT
NP2 = 10240
NPD = 10240
RPT = NP2 // NS
CH = NPD // NS

_mesh = plsc.VectorSubcoreMesh(core_axis_name="c", subcore_axis_name="s")


def _rsqrt16(x):
    i = lax.bitcast_convert_type(x, jnp.int32)
    i = jnp.full((L,), 0x5F3759DF, jnp.int32) - (i >> 1)
    y = lax.bitcast_convert_type(i, jnp.float32)
    for _ in range(3):
        y = y * (1.5 - 0.5 * x * y * y)
    return y


@functools.partial(
    pl.kernel,
    out_type=jax.ShapeDtypeStruct((NC, NPD), jnp.float32),
    mesh=_mesh,
    compiler_params=pltpu.CompilerParams(needs_layout_passes=False),
    scratch_types=[
        pltpu.VMEM((NCHUNK, B), jnp.int32),
        pltpu.VMEM((NCHUNK, B), jnp.float32),
        pltpu.VMEM_SHARED((NPD,), jnp.float32),
    ],
)
def _deg_kernel(colp, ewp, zpd, degpart, idx_v, ew_v, dacc):
    c = lax.axis_index("c")
    s = lax.axis_index("s")
    w = c * NS + s

    @pl.when(s == 0)
    def _():
        pltpu.sync_copy(zpd, dacc)

    pltpu.sync_copy(colp.at[w], idx_v)
    pltpu.sync_copy(ewp.at[w], ew_v)
    plsc.subcore_barrier()

    def body(j, carry):
        pltpu.sync_copy(ew_v.at[j], dacc.at[idx_v.at[j]], add=True)
        return carry

    lax.fori_loop(0, NCHUNK, body, 0)
    plsc.subcore_barrier()
    pltpu.sync_copy(dacc.at[pl.ds(s * CH, CH)],
                    degpart.at[c, pl.ds(s * CH, CH)])


@functools.partial(
    pl.kernel,
    out_type=jax.ShapeDtypeStruct((NW, NCHUNK, B), jnp.float32),
    mesh=_mesh,
    compiler_params=pltpu.CompilerParams(needs_layout_passes=False),
    scratch_types=[
        pltpu.VMEM((NCHUNK, B), jnp.int32),
        pltpu.VMEM((NCHUNK, B), jnp.int32),
        pltpu.VMEM((NCHUNK, B), jnp.float32),
        pltpu.VMEM((NCHUNK, B), jnp.float32),
        pltpu.VMEM((CH,), jnp.float32),
        pltpu.VMEM((CH,), jnp.float32),
        pltpu.VMEM((NPD,), jnp.float32),
        pltpu.VMEM_SHARED((NPD,), jnp.float32),
    ],
)
def _wn_kernel(degpart, rowp, colp, ewp, wn_out,
               idxr, idxc, ew_v, wn_v, d0, d1, dis_v, dis_sh):
    c = lax.axis_index("c")
    s = lax.axis_index("s")
    w = c * NS + s
    pltpu.sync_copy(degpart.at[0, pl.ds(s * CH, CH)], d0)
    pltpu.sync_copy(degpart.at[1, pl.ds(s * CH, CH)], d1)
    for i in range(CH // L):
        sl = pl.ds(i * L, L)
        deg = 1.0 + d0[sl] + d1[sl]
        d0[sl] = _rsqrt16(deg)
    pltpu.sync_copy(d0, dis_sh.at[pl.ds(s * CH, CH)])
    plsc.subcore_barrier()
    pltpu.sync_copy(dis_sh, dis_v)
    pltpu.sync_copy(rowp.at[w], idxr)
    pltpu.sync_copy(colp.at[w], idxc)
    pltpu.sync_copy(ewp.at[w], ew_v)

    def body(j, carry):
        for k in range(B // L):
            sl = pl.ds(k * L, L)
            dr = plsc.load_gather(dis_v, [idxr[j, sl]])
            dc = plsc.load_gather(dis_v, [idxc[j, sl]])
            wn_v[j, sl] = dr * ew_v[j, sl] * dc
        return carry

    lax.fori_loop(0, NCHUNK, body, 0)
    pltpu.sync_copy(wn_v, wn_out.at[w])


@functools.partial(
    pl.kernel,
    out_type=jax.ShapeDtypeStruct((NC, NP2, D), jnp.float32),
    mesh=_mesh,
    compiler_params=pltpu.CompilerParams(needs_layout_passes=False),
    scratch_types=[
        pltpu.VMEM((9, B), jnp.int32),
        pltpu.VMEM((B, D), jnp.float32),
        pltpu.VMEM((B, D), jnp.float32),
        pltpu.VMEM_SHARED((NP2, D), jnp.float32),
        pltpu.SemaphoreType.DMA,
        pltpu.SemaphoreType.DMA,
        pltpu.SemaphoreType.DMA,
        pltpu.SemaphoreType.DMA,
        pltpu.SemaphoreType.DMA,
        pltpu.SemaphoreType.DMA,
        pltpu.SemaphoreType.DMA,
    ],
)
def _spmm_kernel(xp, meta, znd, part,
                 meta_v, rows0, rows1, acc,
                 sm0, sm1, sm2, sg0, sg1, ss0, ss1):
    c = lax.axis_index("c")
    s = lax.axis_index("s")
    w = c * NS + s
    rows = (rows0, rows1)
    sm = (sm0, sm1, sm2)
    sg = (sg0, sg1)
    ss = (ss0, ss1)
    pltpu.sync_copy(znd.at[pl.ds(s * RPT, RPT)], acc.at[pl.ds(s * RPT, RPT)])
    plsc.subcore_barrier()

    def meta_load(j, m):
        pltpu.async_copy(meta.at[w, j], meta_v.at[pl.ds(3 * m, 3)], sm[m])

    def wait_meta(j, m):
        pltpu.make_async_copy(meta.at[w, j], meta_v.at[pl.ds(3 * m, 3)],
                              sm[m]).wait()

    def gather(j, m, p):
        pltpu.async_copy(xp.at[meta_v.at[3 * m]], rows[p], sg[p])

    def wait_gather(p):
        pltpu.make_async_copy(xp.at[meta_v.at[0]], rows[p], sg[p]).wait()

    def scatter(m, p):
        pltpu.async_copy(rows[p], acc.at[meta_v.at[3 * m + 1]], ss[p],
                         add=True)

    def wait_scatter(p):
        pltpu.make_async_copy(rows[p], acc.at[pl.ds(0, B)], ss[p]).wait()

    def scale(m, p):
        rows_v = rows[p]

        def escale(g, icarry):
            wv = lax.bitcast_convert_type(
                meta_v[3 * m + 2, pl.ds(g * L, L)], jnp.float32)
            base = g * L
            for i in range(L):
                tv = jnp.broadcast_to(wv[i], (L,))
                for k in range(D // L):
                    sl = pl.ds(k * L, L)
                    rows_v[base + i, sl] = rows_v[base + i, sl] * tv
            return icarry

        lax.fori_loop(0, B // L, escale, 0)

    meta_load(0, 0)
    meta_load(1, 1)
    wait_meta(0, 0)
    gather(0, 0, 0)
    wait_meta(1, 1)
    gather(1, 1, 1)
    wait_gather(0)
    scale(0, 0)
    scatter(0, 0)
    meta_load(2, 2)

    def step(j, pm1, m, m2, p):
        wait_meta(j + 1, pm1)
        wait_scatter(1 - p)
        gather(j + 1, pm1, 1 - p)
        wait_gather(p)
        scale(m, p)
        scatter(m, p)
        meta_load(j + 2, m2)

    def six(j6, carry):
        jb = 1 + 6 * j6
        for u in range(6):
            step(jb + u, (2 + u) % 3, (1 + u) % 3, u % 3, (1 + u) % 2)
        return carry

    lax.fori_loop(0, (NCHUNK - 2) // 6, six, 0)
    wait_meta(NCHUNK, (NCHUNK) % 3)
    wait_scatter(0)
    wait_gather(1)
    scale((NCHUNK - 1) % 3, 1)
    scatter((NCHUNK - 1) % 3, 1)
    wait_scatter(1)
    plsc.subcore_barrier()
    pltpu.sync_copy(acc.at[pl.ds(s * RPT, RPT)],
                    part.at[c, pl.ds(s * RPT, RPT)])


GB = 10
BR = GB * 128


def _diag_scale(dp, x):
    rr = lax.broadcasted_iota(jnp.int32, (128, 128), 0)
    cc = lax.broadcasted_iota(jnp.int32, (128, 128), 1)
    outs = []
    for g in range(GB):
        iv = 1.0 / (1.0 + dp[g, 0, :] + dp[g, 1, :])
        dg = jnp.where(rr == cc,
                       jnp.broadcast_to(iv[None, :], (128, 128)), 0.0)
        outs.append(jnp.dot(dg, x[g * 128:(g + 1) * 128, :],
                            preferred_element_type=jnp.float32))
    return jnp.concatenate(outs, axis=0)


def _combine_body(dp_ref, p_ref, x_ref, o_ref):
    o_ref[...] = (p_ref[0] + p_ref[1]
                  + _diag_scale(dp_ref[...], x_ref[...]))


_combine_call = pl.pallas_call(
    _combine_body,
    grid=(NP2 // BR,),
    in_specs=[
        pl.BlockSpec((GB, NC, 128), lambda j: (j, 0, 0)),
        pl.BlockSpec((NC, BR, D), lambda j: (0, j, 0)),
        pl.BlockSpec((BR, D), lambda j: (j, 0)),
    ],
    out_specs=pl.BlockSpec((BR, D), lambda j: (j, 0)),
    out_shape=jax.ShapeDtypeStruct((NP2, D), jnp.float32),
)


def _final_body(dp_ref, q_ref, x1_ref, w_ref, b_ref, o_ref):
    h = q_ref[0] + q_ref[1] + _diag_scale(dp_ref[...], x1_ref[...])
    o_ref[...] = lax.dot_general(
        h, w_ref[...], (((1,), (1,)), ((), ())),
        preferred_element_type=jnp.float32) + b_ref[...]


_final_call = pl.pallas_call(
    _final_body,
    grid=(NP2 // BR,),
    in_specs=[
        pl.BlockSpec((GB, NC, 128), lambda j: (j, 0, 0)),
        pl.BlockSpec((NC, BR, D), lambda j: (0, j, 0)),
        pl.BlockSpec((BR, D), lambda j: (j, 0)),
        pl.BlockSpec((CO, D), lambda j: (0, 0)),
        pl.BlockSpec((1, CO), lambda j: (0, 0)),
    ],
    out_specs=pl.BlockSpec((BR, CO), lambda j: (j, 0)),
    out_shape=jax.ShapeDtypeStruct((NP2, CO), jnp.float32),
)


def kernel(data, edge_index, edge_weight, W, b):
    row = edge_index[0]
    col = edge_index[1]
    npad = EPAD - E
    pidx = jnp.arange(npad, dtype=jnp.int32)
    rowp = jnp.concatenate([row, pidx]).reshape(NW, NCHUNK, B)
    colp = jnp.concatenate([col, pidx]).reshape(NW, NCHUNK, B)
    ewp = jnp.concatenate(
        [edge_weight, jnp.zeros((npad,), jnp.float32)]).reshape(NW, NCHUNK, B)
    xp = jnp.concatenate(
        [data, jnp.zeros((NP2 - N, D), jnp.float32)], axis=0)
    zpd = jnp.zeros((NPD,), jnp.float32)
    znd = jnp.zeros((NP2, D), jnp.float32)

    degpart = _deg_kernel(colp, ewp, zpd)
    wn = _wn_kernel(degpart, rowp, colp, ewp)
    dpr = degpart.reshape(NC, NPD // 128, 128).transpose(1, 0, 2)

    wni = lax.bitcast_convert_type(wn, jnp.int32)
    meta = jnp.stack([colp, rowp, wni], axis=2)
    meta = jnp.concatenate(
        [meta, jnp.zeros((NW, 1, 3, B), jnp.int32)], axis=1)

    part1 = _spmm_kernel(xp, meta, znd)
    x1 = _combine_call(dpr, part1, xp)
    part2 = _spmm_kernel(x1, meta, znd)
    out = _final_call(dpr, part2, x1, W, b.reshape(1, CO))
    return out[:N]


# speedup vs baseline: 28.0139x; 1.0379x over previous
import functools

import jax
import jax.numpy as jnp
from jax import lax
from jax.experimental import pallas as pl
from jax.experimental.pallas import tpu as pltpu
from jax.experimental.pallas import tpu_sc as plsc

N = 10000
E = 320000
D = 128
CO = 128
NC = 2
NS = 16
NW = NC * NS
L = 16

B = 128
NCHUNK = 80
EPT = B * NCHUNK
EPAD = NW * EPT
NP2 = 10240
NPD = 10240
NROWS = N
RPT = 632
RLAST = NROWS - 15 * RPT
CH = NPD // NS

_mesh = plsc.VectorSubcoreMesh(core_axis_name="c", subcore_axis_name="s")


def _rsqrt16(x):
    i = lax.bitcast_convert_type(x, jnp.int32)
    i = jnp.full((L,), 0x5F3759DF, jnp.int32) - (i >> 1)
    y = lax.bitcast_convert_type(i, jnp.float32)
    for _ in range(3):
        y = y * (1.5 - 0.5 * x * y * y)
    return y


@functools.partial(
    pl.kernel,
    out_type=jax.ShapeDtypeStruct((NC, NPD), jnp.float32),
    mesh=_mesh,
    compiler_params=pltpu.CompilerParams(needs_layout_passes=False),
    scratch_types=[
        pltpu.VMEM((NCHUNK, B), jnp.int32),
        pltpu.VMEM((NCHUNK, B), jnp.float32),
        pltpu.VMEM_SHARED((NPD,), jnp.float32),
    ],
)
def _deg_kernel(colp, ewp, zpd, degpart, idx_v, ew_v, dacc):
    c = lax.axis_index("c")
    s = lax.axis_index("s")
    w = c * NS + s

    @pl.when(s == 0)
    def _():
        pltpu.sync_copy(zpd, dacc)

    pltpu.sync_copy(colp.at[w], idx_v)
    pltpu.sync_copy(ewp.at[w], ew_v)
    plsc.subcore_barrier()

    def body(j, carry):
        pltpu.sync_copy(ew_v.at[j], dacc.at[idx_v.at[j]], add=True)
        return carry

    lax.fori_loop(0, NCHUNK, body, 0)
    plsc.subcore_barrier()
    pltpu.sync_copy(dacc.at[pl.ds(s * CH, CH)],
                    degpart.at[c, pl.ds(s * CH, CH)])


@functools.partial(
    pl.kernel,
    out_type=jax.ShapeDtypeStruct((NW, NCHUNK, B), jnp.float32),
    mesh=_mesh,
    compiler_params=pltpu.CompilerParams(needs_layout_passes=False),
    scratch_types=[
        pltpu.VMEM((NCHUNK, B), jnp.int32),
        pltpu.VMEM((NCHUNK, B), jnp.int32),
        pltpu.VMEM((NCHUNK, B), jnp.float32),
        pltpu.VMEM((NCHUNK, B), jnp.float32),
        pltpu.VMEM((CH,), jnp.float32),
        pltpu.VMEM((CH,), jnp.float32),
        pltpu.VMEM((NPD,), jnp.float32),
        pltpu.VMEM_SHARED((NPD,), jnp.float32),
    ],
)
def _wn_kernel(degpart, rowp, colp, ewp, wn_out,
               idxr, idxc, ew_v, wn_v, d0, d1, dis_v, dis_sh):
    c = lax.axis_index("c")
    s = lax.axis_index("s")
    w = c * NS + s
    pltpu.sync_copy(degpart.at[0, pl.ds(s * CH, CH)], d0)
    pltpu.sync_copy(degpart.at[1, pl.ds(s * CH, CH)], d1)
    for i in range(CH // L):
        sl = pl.ds(i * L, L)
        deg = 1.0 + d0[sl] + d1[sl]
        d0[sl] = _rsqrt16(deg)
    pltpu.sync_copy(d0, dis_sh.at[pl.ds(s * CH, CH)])
    plsc.subcore_barrier()
    pltpu.sync_copy(dis_sh, dis_v)
    pltpu.sync_copy(rowp.at[w], idxr)
    pltpu.sync_copy(colp.at[w], idxc)
    pltpu.sync_copy(ewp.at[w], ew_v)

    def body(j, carry):
        for k in range(B // L):
            sl = pl.ds(k * L, L)
            dr = plsc.load_gather(dis_v, [idxr[j, sl]])
            dc = plsc.load_gather(dis_v, [idxc[j, sl]])
            wn_v[j, sl] = dr * ew_v[j, sl] * dc
        return carry

    lax.fori_loop(0, NCHUNK, body, 0)
    pltpu.sync_copy(wn_v, wn_out.at[w])


@functools.partial(
    pl.kernel,
    out_type=jax.ShapeDtypeStruct((NC, NP2, D), jnp.float32),
    mesh=_mesh,
    compiler_params=pltpu.CompilerParams(
        needs_layout_passes=False, internal_scratch_in_bytes=16384),
    scratch_types=[
        pltpu.VMEM((8, B), jnp.int32),
        pltpu.VMEM((B, D), jnp.float32),
        pltpu.VMEM((B, D), jnp.float32),
        pltpu.VMEM((B, D), jnp.float32),
        pltpu.VMEM_SHARED((NROWS, D), jnp.float32),
        pltpu.SemaphoreType.DMA,
        pltpu.SemaphoreType.DMA,
        pltpu.SemaphoreType.DMA,
        pltpu.SemaphoreType.DMA,
        pltpu.SemaphoreType.DMA,
        pltpu.SemaphoreType.DMA,
        pltpu.SemaphoreType.DMA,
        pltpu.SemaphoreType.DMA,
    ],
)
def _spmm_kernel(xp, meta, part,
                 meta_v, rows0, rows1, rows2, acc,
                 sm0, sm1, sg0, sg1, sg2, ss0, ss1, ss2):
    c = lax.axis_index("c")
    s = lax.axis_index("s")
    w = c * NS + s
    rows = (rows0, rows1, rows2)
    sm = (sm0, sm1)
    sg = (sg0, sg1, sg2)
    ss = (ss0, ss1, ss2)
    def zrow(r, carry):
        for k in range(D // L):
            rows0[r, pl.ds(k * L, L)] = jnp.zeros((L,), jnp.float32)
        return carry

    lax.fori_loop(0, B, zrow, 0)
    base = s * RPT
    for k in range(4):
        pltpu.sync_copy(rows0, acc.at[pl.ds(base + k * B, B)])

    @pl.when(s < NS - 1)
    def _():
        pltpu.sync_copy(rows0.at[pl.ds(0, RPT - 4 * B)],
                        acc.at[pl.ds(base + 4 * B, RPT - 4 * B)])

    @pl.when(s == NS - 1)
    def _():
        pltpu.sync_copy(rows0.at[pl.ds(0, RLAST - 4 * B)],
                        acc.at[pl.ds(base + 4 * B, RLAST - 4 * B)])

    plsc.subcore_barrier()

    def meta_load(j, m):
        pltpu.async_copy(meta.at[w, j], meta_v.at[pl.ds(3 * m, 3)], sm[m])

    def wait_meta(j, m):
        pltpu.make_async_copy(meta.at[w, j], meta_v.at[pl.ds(3 * m, 3)],
                              sm[m]).wait()

    def gather(j, m, p):
        pltpu.async_copy(xp.at[meta_v.at[3 * m]], rows[p], sg[p])

    def wait_gather(p):
        pltpu.make_async_copy(xp.at[meta_v.at[0]], rows[p], sg[p]).wait()

    def scatter(m, p):
        pltpu.async_copy(rows[p], acc.at[meta_v.at[6 + m]], ss[p],
                         add=True)

    def wait_scatter(p):
        pltpu.make_async_copy(rows[p], acc.at[pl.ds(0, B)], ss[p]).wait()

    def scale(m, p):
        rows_v = rows[p]

        def escale(g, icarry):
            wv = lax.bitcast_convert_type(
                meta_v[3 * m + 2, pl.ds(g * L, L)], jnp.float32)
            base = g * L
            for i in range(L):
                tv = jnp.broadcast_to(wv[i], (L,))
                for k in range(D // L):
                    sl = pl.ds(k * L, L)
                    rows_v[base + i, sl] = rows_v[base + i, sl] * tv
            return icarry

        lax.fori_loop(0, B // L, escale, 0)
        for k in range(B // L):
            sl = pl.ds(k * L, L)
            meta_v[6 + m, sl] = meta_v[3 * m + 1, sl]

    def six(i6, carry):
        jb = 6 * i6 - 2
        for u in range(6):
            j = jb + u
            m2 = u % 2
            n2 = (u + 1) % 2
            m3 = (u + 1) % 3
            n3 = (u + 2) % 3

            @pl.when((j >= -1) & (j <= NCHUNK - 2))
            def _():
                wait_meta(j + 1, n2)

            @pl.when(j >= 2)
            def _():
                wait_scatter(n3)

            @pl.when((j >= -1) & (j <= NCHUNK - 2))
            def _():
                gather(j + 1, n2, n3)

            @pl.when((j >= 0) & (j <= NCHUNK - 1))
            def _():
                wait_gather(m3)
                scale(m2, m3)
                scatter(m2, m3)

            @pl.when(j <= NCHUNK - 3)
            def _():
                meta_load(j + 2, m2)
        return carry

    lax.fori_loop(0, 14, six, 0)
    plsc.subcore_barrier()

    @pl.when(s < NS - 1)
    def _():
        pltpu.sync_copy(acc.at[pl.ds(s * RPT, RPT)],
                        part.at[c, pl.ds(s * RPT, RPT)])

    @pl.when(s == NS - 1)
    def _():
        pltpu.sync_copy(acc.at[pl.ds(15 * RPT, RLAST)],
                        part.at[c, pl.ds(15 * RPT, RLAST)])


GB = 10
BR = GB * 128


def _diag_scale(dp, x):
    rr = lax.broadcasted_iota(jnp.int32, (128, 128), 0)
    cc = lax.broadcasted_iota(jnp.int32, (128, 128), 1)
    outs = []
    for g in range(GB):
        iv = 1.0 / (1.0 + dp[g, 0, :] + dp[g, 1, :])
        dg = jnp.where(rr == cc,
                       jnp.broadcast_to(iv[None, :], (128, 128)), 0.0)
        outs.append(jnp.dot(dg, x[g * 128:(g + 1) * 128, :],
                            preferred_element_type=jnp.float32))
    return jnp.concatenate(outs, axis=0)


def _combine_body(dp_ref, p_ref, x_ref, o_ref):
    o_ref[...] = (p_ref[0] + p_ref[1]
                  + _diag_scale(dp_ref[...], x_ref[...]))


_combine_call = pl.pallas_call(
    _combine_body,
    grid=(NP2 // BR,),
    in_specs=[
        pl.BlockSpec((GB, NC, 128), lambda j: (j, 0, 0)),
        pl.BlockSpec((NC, BR, D), lambda j: (0, j, 0)),
        pl.BlockSpec((BR, D), lambda j: (j, 0)),
    ],
    out_specs=pl.BlockSpec((BR, D), lambda j: (j, 0)),
    out_shape=jax.ShapeDtypeStruct((NP2, D), jnp.float32),
)


def _final_body(dp_ref, q_ref, x1_ref, w_ref, b_ref, o_ref):
    h = q_ref[0] + q_ref[1] + _diag_scale(dp_ref[...], x1_ref[...])
    o_ref[...] = lax.dot_general(
        h, w_ref[...], (((1,), (1,)), ((), ())),
        preferred_element_type=jnp.float32) + b_ref[...]


_final_call = pl.pallas_call(
    _final_body,
    grid=(NP2 // BR,),
    in_specs=[
        pl.BlockSpec((GB, NC, 128), lambda j: (j, 0, 0)),
        pl.BlockSpec((NC, BR, D), lambda j: (0, j, 0)),
        pl.BlockSpec((BR, D), lambda j: (j, 0)),
        pl.BlockSpec((CO, D), lambda j: (0, 0)),
        pl.BlockSpec((1, CO), lambda j: (0, 0)),
    ],
    out_specs=pl.BlockSpec((BR, CO), lambda j: (j, 0)),
    out_shape=jax.ShapeDtypeStruct((NP2, CO), jnp.float32),
)


def kernel(data, edge_index, edge_weight, W, b):
    row = edge_index[0]
    col = edge_index[1]
    npad = EPAD - E
    pidx = jnp.arange(npad, dtype=jnp.int32)
    rowp = jnp.concatenate([row, pidx]).reshape(NW, NCHUNK, B)
    colp = jnp.concatenate([col, pidx]).reshape(NW, NCHUNK, B)
    ewp = jnp.concatenate(
        [edge_weight, jnp.zeros((npad,), jnp.float32)]).reshape(NW, NCHUNK, B)
    xp = jnp.concatenate(
        [data, jnp.zeros((NP2 - N, D), jnp.float32)], axis=0)
    zpd = jnp.zeros((NPD,), jnp.float32)

    degpart = _deg_kernel(colp, ewp, zpd)
    wn = _wn_kernel(degpart, rowp, colp, ewp)
    dpr = degpart.reshape(NC, NPD // 128, 128).transpose(1, 0, 2)

    wni = lax.bitcast_convert_type(wn, jnp.int32)
    meta = jnp.stack([colp, rowp, wni], axis=2)
    meta = jnp.concatenate(
        [meta, jnp.zeros((NW, 1, 3, B), jnp.int32)], axis=1)

    part1 = _spmm_kernel(xp, meta)
    x1 = _combine_call(dpr, part1, xp)
    part2 = _spmm_kernel(x1, meta)
    out = _final_call(dpr, part2, x1, W, b.reshape(1, CO))
    return out[:N]


# speedup vs baseline: 28.6014x; 1.0210x over previous
import functools

import jax
import jax.numpy as jnp
from jax import lax
from jax.experimental import pallas as pl
from jax.experimental.pallas import tpu as pltpu
from jax.experimental.pallas import tpu_sc as plsc

N = 10000
E = 320000
D = 128
CO = 128
NC = 2
NS = 16
NW = NC * NS
L = 16

B = 128
NCHUNK = 80
EPT = B * NCHUNK
EPAD = NW * EPT
NP2 = 10240
NPD = 10240
NROWS = N
RPT = 632
RLAST = NROWS - 15 * RPT
CH = NPD // NS

_mesh = plsc.VectorSubcoreMesh(core_axis_name="c", subcore_axis_name="s")


def _rsqrt16(x):
    i = lax.bitcast_convert_type(x, jnp.int32)
    i = jnp.full((L,), 0x5F3759DF, jnp.int32) - (i >> 1)
    y = lax.bitcast_convert_type(i, jnp.float32)
    for _ in range(3):
        y = y * (1.5 - 0.5 * x * y * y)
    return y


@functools.partial(
    pl.kernel,
    out_type=jax.ShapeDtypeStruct((NC, NPD), jnp.float32),
    mesh=_mesh,
    compiler_params=pltpu.CompilerParams(needs_layout_passes=False),
    scratch_types=[
        pltpu.VMEM((NCHUNK, B), jnp.int32),
        pltpu.VMEM((NCHUNK, B), jnp.float32),
        pltpu.VMEM_SHARED((NPD,), jnp.float32),
        pltpu.SemaphoreType.DMA,
    ],
)
def _deg_kernel(colp, ewp, zpd, degpart, idx_v, ew_v, dacc, sd):
    c = lax.axis_index("c")
    s = lax.axis_index("s")
    w = c * NS + s

    @pl.when(s == 0)
    def _():
        pltpu.sync_copy(zpd, dacc)

    pltpu.sync_copy(colp.at[w], idx_v)
    pltpu.sync_copy(ewp.at[w], ew_v)
    plsc.subcore_barrier()

    def body(j, carry):
        pltpu.async_copy(ew_v.at[j], dacc.at[idx_v.at[j]], sd, add=True)

        @pl.when(j >= 4)
        def _():
            pltpu.make_async_copy(ew_v.at[0], dacc.at[pl.ds(0, B)],
                                  sd).wait()

        return carry

    lax.fori_loop(0, NCHUNK, body, 0)
    for _ in range(4):
        pltpu.make_async_copy(ew_v.at[0], dacc.at[pl.ds(0, B)], sd).wait()
    plsc.subcore_barrier()
    pltpu.sync_copy(dacc.at[pl.ds(s * CH, CH)],
                    degpart.at[c, pl.ds(s * CH, CH)])


@functools.partial(
    pl.kernel,
    out_type=jax.ShapeDtypeStruct((NW, NCHUNK, B), jnp.float32),
    mesh=_mesh,
    compiler_params=pltpu.CompilerParams(needs_layout_passes=False),
    scratch_types=[
        pltpu.VMEM((NCHUNK, B), jnp.int32),
        pltpu.VMEM((NCHUNK, B), jnp.int32),
        pltpu.VMEM((NCHUNK, B), jnp.float32),
        pltpu.VMEM((NCHUNK, B), jnp.float32),
        pltpu.VMEM((CH,), jnp.float32),
        pltpu.VMEM((CH,), jnp.float32),
        pltpu.VMEM((NPD,), jnp.float32),
        pltpu.VMEM_SHARED((NPD,), jnp.float32),
    ],
)
def _wn_kernel(degpart, rowp, colp, ewp, wn_out,
               idxr, idxc, ew_v, wn_v, d0, d1, dis_v, dis_sh):
    c = lax.axis_index("c")
    s = lax.axis_index("s")
    w = c * NS + s
    pltpu.sync_copy(degpart.at[0, pl.ds(s * CH, CH)], d0)
    pltpu.sync_copy(degpart.at[1, pl.ds(s * CH, CH)], d1)
    for i in range(CH // L):
        sl = pl.ds(i * L, L)
        deg = 1.0 + d0[sl] + d1[sl]
        d0[sl] = _rsqrt16(deg)
    pltpu.sync_copy(d0, dis_sh.at[pl.ds(s * CH, CH)])
    plsc.subcore_barrier()
    pltpu.sync_copy(dis_sh, dis_v)
    pltpu.sync_copy(rowp.at[w], idxr)
    pltpu.sync_copy(colp.at[w], idxc)
    pltpu.sync_copy(ewp.at[w], ew_v)

    def body(j, carry):
        for k in range(B // L):
            sl = pl.ds(k * L, L)
            dr = plsc.load_gather(dis_v, [idxr[j, sl]])
            dc = plsc.load_gather(dis_v, [idxc[j, sl]])
            wn_v[j, sl] = dr * ew_v[j, sl] * dc
        return carry

    lax.fori_loop(0, NCHUNK, body, 0)
    pltpu.sync_copy(wn_v, wn_out.at[w])


@functools.partial(
    pl.kernel,
    out_type=jax.ShapeDtypeStruct((NC, NP2, D), jnp.float32),
    mesh=_mesh,
    compiler_params=pltpu.CompilerParams(
        needs_layout_passes=False, internal_scratch_in_bytes=16384),
    scratch_types=[
        pltpu.VMEM((8, B), jnp.int32),
        pltpu.VMEM((B, D), jnp.float32),
        pltpu.VMEM((B, D), jnp.float32),
        pltpu.VMEM((B, D), jnp.float32),
        pltpu.VMEM_SHARED((NROWS, D), jnp.float32),
        pltpu.SemaphoreType.DMA,
        pltpu.SemaphoreType.DMA,
        pltpu.SemaphoreType.DMA,
        pltpu.SemaphoreType.DMA,
        pltpu.SemaphoreType.DMA,
        pltpu.SemaphoreType.DMA,
        pltpu.SemaphoreType.DMA,
        pltpu.SemaphoreType.DMA,
    ],
)
def _spmm_kernel(xp, meta, part,
                 meta_v, rows0, rows1, rows2, acc,
                 sm0, sm1, sg0, sg1, sg2, ss0, ss1, ss2):
    c = lax.axis_index("c")
    s = lax.axis_index("s")
    w = c * NS + s
    rows = (rows0, rows1, rows2)
    sm = (sm0, sm1)
    sg = (sg0, sg1, sg2)
    ss = (ss0, ss1, ss2)
    def zrow(r, carry):
        for k in range(D // L):
            rows0[r, pl.ds(k * L, L)] = jnp.zeros((L,), jnp.float32)
        return carry

    lax.fori_loop(0, B, zrow, 0)
    base = s * RPT
    for k in range(4):
        pltpu.sync_copy(rows0, acc.at[pl.ds(base + k * B, B)])

    @pl.when(s < NS - 1)
    def _():
        pltpu.sync_copy(rows0.at[pl.ds(0, RPT - 4 * B)],
                        acc.at[pl.ds(base + 4 * B, RPT - 4 * B)])

    @pl.when(s == NS - 1)
    def _():
        pltpu.sync_copy(rows0.at[pl.ds(0, RLAST - 4 * B)],
                        acc.at[pl.ds(base + 4 * B, RLAST - 4 * B)])

    plsc.subcore_barrier()

    def meta_load(j, m):
        pltpu.async_copy(meta.at[w, j], meta_v.at[pl.ds(3 * m, 3)], sm[m])

    def wait_meta(j, m):
        pltpu.make_async_copy(meta.at[w, j], meta_v.at[pl.ds(3 * m, 3)],
                              sm[m]).wait()

    def gather(j, m, p):
        pltpu.async_copy(xp.at[meta_v.at[3 * m]], rows[p], sg[p])

    def wait_gather(p):
        pltpu.make_async_copy(xp.at[meta_v.at[0]], rows[p], sg[p]).wait()

    def scatter(m, p):
        pltpu.async_copy(rows[p], acc.at[meta_v.at[6 + m]], ss[p],
                         add=True)

    def wait_scatter(p):
        pltpu.make_async_copy(rows[p], acc.at[pl.ds(0, B)], ss[p]).wait()

    def scale(m, p):
        rows_v = rows[p]

        def escale(g, icarry):
            wv = lax.bitcast_convert_type(
                meta_v[3 * m + 2, pl.ds(g * L, L)], jnp.float32)
            base = g * L
            for i in range(L):
                tv = jnp.broadcast_to(wv[i], (L,))
                for k in range(D // L):
                    sl = pl.ds(k * L, L)
                    rows_v[base + i, sl] = rows_v[base + i, sl] * tv
            return icarry

        lax.fori_loop(0, B // L, escale, 0)
        for k in range(B // L):
            sl = pl.ds(k * L, L)
            meta_v[6 + m, sl] = meta_v[3 * m + 1, sl]

    def six(i6, carry):
        jb = 6 * i6 - 2
        for u in range(6):
            j = jb + u
            m2 = u % 2
            n2 = (u + 1) % 2
            m3 = (u + 1) % 3
            n3 = (u + 2) % 3

            @pl.when((j >= -1) & (j <= NCHUNK - 2))
            def _():
                wait_meta(j + 1, n2)

            @pl.when(j >= 2)
            def _():
                wait_scatter(n3)

            @pl.when((j >= -1) & (j <= NCHUNK - 2))
            def _():
                gather(j + 1, n2, n3)

            @pl.when((j >= 0) & (j <= NCHUNK - 1))
            def _():
                wait_gather(m3)
                scale(m2, m3)
                scatter(m2, m3)

            @pl.when(j <= NCHUNK - 3)
            def _():
                meta_load(j + 2, m2)
        return carry

    lax.fori_loop(0, 14, six, 0)
    plsc.subcore_barrier()

    @pl.when(s < NS - 1)
    def _():
        pltpu.sync_copy(acc.at[pl.ds(s * RPT, RPT)],
                        part.at[c, pl.ds(s * RPT, RPT)])

    @pl.when(s == NS - 1)
    def _():
        pltpu.sync_copy(acc.at[pl.ds(15 * RPT, RLAST)],
                        part.at[c, pl.ds(15 * RPT, RLAST)])


GB = 8
BR = GB * 128


def _diag_scale(dp, x):
    rr = lax.broadcasted_iota(jnp.int32, (128, 128), 0)
    cc = lax.broadcasted_iota(jnp.int32, (128, 128), 1)
    outs = []
    for g in range(GB):
        iv = 1.0 / (1.0 + dp[0, g, :] + dp[1, g, :])
        dg = jnp.where(rr == cc,
                       jnp.broadcast_to(iv[None, :], (128, 128)), 0.0)
        outs.append(jnp.dot(dg, x[g * 128:(g + 1) * 128, :],
                            preferred_element_type=jnp.float32))
    return jnp.concatenate(outs, axis=0)


def _combine_body(dp_ref, p_ref, x_ref, o_ref):
    o_ref[...] = (p_ref[0] + p_ref[1]
                  + _diag_scale(dp_ref[...], x_ref[...]))


_combine_call = pl.pallas_call(
    _combine_body,
    grid=(NP2 // BR,),
    in_specs=[
        pl.BlockSpec((NC, GB, 128), lambda j: (0, j, 0)),
        pl.BlockSpec((NC, BR, D), lambda j: (0, j, 0)),
        pl.BlockSpec((BR, D), lambda j: (j, 0)),
    ],
    out_specs=pl.BlockSpec((BR, D), lambda j: (j, 0)),
    out_shape=jax.ShapeDtypeStruct((NP2, D), jnp.float32),
)


def _final_body(dp_ref, q_ref, x1_ref, w_ref, b_ref, o_ref):
    h = q_ref[0] + q_ref[1] + _diag_scale(dp_ref[...], x1_ref[...])
    o_ref[...] = lax.dot_general(
        h, w_ref[...], (((1,), (1,)), ((), ())),
        preferred_element_type=jnp.float32) + b_ref[...]


_final_call = pl.pallas_call(
    _final_body,
    grid=(NP2 // BR,),
    in_specs=[
        pl.BlockSpec((NC, GB, 128), lambda j: (0, j, 0)),
        pl.BlockSpec((NC, BR, D), lambda j: (0, j, 0)),
        pl.BlockSpec((BR, D), lambda j: (j, 0)),
        pl.BlockSpec((CO, D), lambda j: (0, 0)),
        pl.BlockSpec((1, CO), lambda j: (0, 0)),
    ],
    out_specs=pl.BlockSpec((BR, CO), lambda j: (j, 0)),
    out_shape=jax.ShapeDtypeStruct((N, CO), jnp.float32),
)


def kernel(data, edge_index, edge_weight, W, b):
    row = edge_index[0]
    col = edge_index[1]
    npad = EPAD - E
    pidx = jnp.arange(npad, dtype=jnp.int32)
    rowp = jnp.concatenate([row, pidx]).reshape(NW, NCHUNK, B)
    colp = jnp.concatenate([col, pidx]).reshape(NW, NCHUNK, B)
    ewp = jnp.concatenate(
        [edge_weight, jnp.zeros((npad,), jnp.float32)]).reshape(NW, NCHUNK, B)
    xp = jnp.concatenate(
        [data, jnp.zeros((NP2 - N, D), jnp.float32)], axis=0)
    zpd = jnp.zeros((NPD,), jnp.float32)

    degpart = _deg_kernel(colp, ewp, zpd)
    wn = _wn_kernel(degpart, rowp, colp, ewp)
    dpr = degpart.reshape(NC, NPD // 128, 128)

    wni = lax.bitcast_convert_type(wn, jnp.int32)
    meta = jnp.stack([colp, rowp, wni], axis=2)
    meta = jnp.concatenate(
        [meta, jnp.zeros((NW, 1, 3, B), jnp.int32)], axis=1)

    part1 = _spmm_kernel(xp, meta)
    x1 = _combine_call(dpr, part1, xp)
    part2 = _spmm_kernel(x1, meta)
    return _final_call(dpr, part2, x1, W, b.reshape(1, CO))


# speedup vs baseline: 28.8232x; 1.0078x over previous
import functools

import jax
import jax.numpy as jnp
from jax import lax
from jax.experimental import pallas as pl
from jax.experimental.pallas import tpu as pltpu
from jax.experimental.pallas import tpu_sc as plsc

N = 10000
E = 320000
D = 128
CO = 128
NC = 2
NS = 16
NW = NC * NS
L = 16

B = 128
NCHUNK = 80
EPT = B * NCHUNK
EPAD = NW * EPT
NP2 = 10240
NPD = 10240
NROWS = N
RPT = 632
RLAST = NROWS - 15 * RPT
CH = NPD // NS

_mesh = plsc.VectorSubcoreMesh(core_axis_name="c", subcore_axis_name="s")


def _rsqrt16(x):
    i = lax.bitcast_convert_type(x, jnp.int32)
    i = jnp.full((L,), 0x5F3759DF, jnp.int32) - (i >> 1)
    y = lax.bitcast_convert_type(i, jnp.float32)
    for _ in range(3):
        y = y * (1.5 - 0.5 * x * y * y)
    return y


@functools.partial(
    pl.kernel,
    out_type=jax.ShapeDtypeStruct((NC, NPD), jnp.float32),
    mesh=_mesh,
    compiler_params=pltpu.CompilerParams(needs_layout_passes=False),
    scratch_types=[
        pltpu.VMEM((NCHUNK, B), jnp.int32),
        pltpu.VMEM((NCHUNK, B), jnp.float32),
        pltpu.VMEM_SHARED((NPD,), jnp.float32),
        pltpu.SemaphoreType.DMA,
    ],
)
def _deg_kernel(colp, ewp, zpd, degpart, idx_v, ew_v, dacc, sd):
    c = lax.axis_index("c")
    s = lax.axis_index("s")
    w = c * NS + s

    @pl.when(s == 0)
    def _():
        pltpu.sync_copy(zpd, dacc)

    pltpu.sync_copy(colp.at[w], idx_v)
    pltpu.sync_copy(ewp.at[w], ew_v)
    plsc.subcore_barrier()

    def body(j, carry):
        pltpu.async_copy(ew_v.at[j], dacc.at[idx_v.at[j]], sd, add=True)

        @pl.when(j >= 4)
        def _():
            pltpu.make_async_copy(ew_v.at[0], dacc.at[pl.ds(0, B)],
                                  sd).wait()

        return carry

    lax.fori_loop(0, NCHUNK, body, 0)
    for _ in range(4):
        pltpu.make_async_copy(ew_v.at[0], dacc.at[pl.ds(0, B)], sd).wait()
    plsc.subcore_barrier()
    pltpu.sync_copy(dacc.at[pl.ds(s * CH, CH)],
                    degpart.at[c, pl.ds(s * CH, CH)])


@functools.partial(
    pl.kernel,
    out_type=jax.ShapeDtypeStruct((NW, NCHUNK, B), jnp.float32),
    mesh=_mesh,
    compiler_params=pltpu.CompilerParams(needs_layout_passes=False),
    scratch_types=[
        pltpu.VMEM((NCHUNK, B), jnp.int32),
        pltpu.VMEM((NCHUNK, B), jnp.int32),
        pltpu.VMEM((NCHUNK, B), jnp.float32),
        pltpu.VMEM((NCHUNK, B), jnp.float32),
        pltpu.VMEM((CH,), jnp.float32),
        pltpu.VMEM((CH,), jnp.float32),
        pltpu.VMEM((NPD,), jnp.float32),
        pltpu.VMEM_SHARED((NPD,), jnp.float32),
    ],
)
def _wn_kernel(degpart, rowp, colp, ewp, wn_out,
               idxr, idxc, ew_v, wn_v, d0, d1, dis_v, dis_sh):
    c = lax.axis_index("c")
    s = lax.axis_index("s")
    w = c * NS + s
    pltpu.sync_copy(degpart.at[0, pl.ds(s * CH, CH)], d0)
    pltpu.sync_copy(degpart.at[1, pl.ds(s * CH, CH)], d1)
    for i in range(CH // L):
        sl = pl.ds(i * L, L)
        deg = 1.0 + d0[sl] + d1[sl]
        d0[sl] = _rsqrt16(deg)
    pltpu.sync_copy(d0, dis_sh.at[pl.ds(s * CH, CH)])
    plsc.subcore_barrier()
    pltpu.sync_copy(dis_sh, dis_v)
    pltpu.sync_copy(rowp.at[w], idxr)
    pltpu.sync_copy(colp.at[w], idxc)
    pltpu.sync_copy(ewp.at[w], ew_v)

    def body(j, carry):
        for k in range(B // L):
            sl = pl.ds(k * L, L)
            dr = plsc.load_gather(dis_v, [idxr[j, sl]])
            dc = plsc.load_gather(dis_v, [idxc[j, sl]])
            wn_v[j, sl] = dr * ew_v[j, sl] * dc
        return carry

    lax.fori_loop(0, NCHUNK, body, 0)
    pltpu.sync_copy(wn_v, wn_out.at[w])


@functools.partial(
    pl.kernel,
    out_type=jax.ShapeDtypeStruct((NC, NP2, D), jnp.float32),
    mesh=_mesh,
    compiler_params=pltpu.CompilerParams(
        needs_layout_passes=False, internal_scratch_in_bytes=16384),
    scratch_types=[
        pltpu.VMEM((8, B), jnp.int32),
        pltpu.VMEM((B, D), jnp.float32),
        pltpu.VMEM((B, D), jnp.float32),
        pltpu.VMEM((B, D), jnp.float32),
        pltpu.VMEM_SHARED((NROWS, D), jnp.float32),
        pltpu.SemaphoreType.DMA,
        pltpu.SemaphoreType.DMA,
        pltpu.SemaphoreType.DMA,
        pltpu.SemaphoreType.DMA,
        pltpu.SemaphoreType.DMA,
        pltpu.SemaphoreType.DMA,
        pltpu.SemaphoreType.DMA,
        pltpu.SemaphoreType.DMA,
    ],
)
def _spmm_kernel(xp, meta, part,
                 meta_v, rows0, rows1, rows2, acc,
                 sm0, sm1, sg0, sg1, sg2, ss0, ss1, ss2):
    c = lax.axis_index("c")
    s = lax.axis_index("s")
    w = c * NS + s
    rows = (rows0, rows1, rows2)
    sm = (sm0, sm1)
    sg = (sg0, sg1, sg2)
    ss = (ss0, ss1, ss2)
    def zrow(r, carry):
        for k in range(D // L):
            rows0[r, pl.ds(k * L, L)] = jnp.zeros((L,), jnp.float32)
        return carry

    lax.fori_loop(0, B, zrow, 0)
    base = s * RPT
    for k in range(4):
        pltpu.sync_copy(rows0, acc.at[pl.ds(base + k * B, B)])

    @pl.when(s < NS - 1)
    def _():
        pltpu.sync_copy(rows0.at[pl.ds(0, RPT - 4 * B)],
                        acc.at[pl.ds(base + 4 * B, RPT - 4 * B)])

    @pl.when(s == NS - 1)
    def _():
        pltpu.sync_copy(rows0.at[pl.ds(0, RLAST - 4 * B)],
                        acc.at[pl.ds(base + 4 * B, RLAST - 4 * B)])

    plsc.subcore_barrier()

    def meta_load(j, m):
        pltpu.async_copy(meta.at[w, j], meta_v.at[pl.ds(3 * m, 3)], sm[m])

    def wait_meta(j, m):
        pltpu.make_async_copy(meta.at[w, j], meta_v.at[pl.ds(3 * m, 3)],
                              sm[m]).wait()

    def gather(j, m, p):
        pltpu.async_copy(xp.at[meta_v.at[3 * m]], rows[p], sg[p])

    def wait_gather(p):
        pltpu.make_async_copy(xp.at[meta_v.at[0]], rows[p], sg[p]).wait()

    def scatter(m, p):
        pltpu.async_copy(rows[p], acc.at[meta_v.at[6 + m]], ss[p],
                         add=True)

    def wait_scatter(p):
        pltpu.make_async_copy(rows[p], acc.at[pl.ds(0, B)], ss[p]).wait()

    def scale(m, p):
        rows_v = rows[p]

        def escale(g, icarry):
            wv = lax.bitcast_convert_type(
                meta_v[3 * m + 2, pl.ds(g * L, L)], jnp.float32)
            base = g * L
            for i in range(L):
                tv = jnp.broadcast_to(wv[i], (L,))
                for k in range(D // L):
                    sl = pl.ds(k * L, L)
                    rows_v[base + i, sl] = rows_v[base + i, sl] * tv
            return icarry

        lax.fori_loop(0, B // L, escale, 0)
        for k in range(B // L):
            sl = pl.ds(k * L, L)
            meta_v[6 + m, sl] = meta_v[3 * m + 1, sl]

    def six(i6, carry):
        jb = 6 * i6 - 2
        for u in range(6):
            j = jb + u
            m2 = u % 2
            n2 = (u + 1) % 2
            m3 = (u + 1) % 3
            n3 = (u + 2) % 3

            @pl.when((j >= -1) & (j <= NCHUNK - 2))
            def _():
                wait_meta(j + 1, n2)

            @pl.when(j >= 2)
            def _():
                wait_scatter(n3)

            @pl.when((j >= -1) & (j <= NCHUNK - 2))
            def _():
                gather(j + 1, n2, n3)

            @pl.when((j >= 0) & (j <= NCHUNK - 1))
            def _():
                wait_gather(m3)
                scale(m2, m3)
                scatter(m2, m3)

            @pl.when(j <= NCHUNK - 3)
            def _():
                meta_load(j + 2, m2)
        return carry

    lax.fori_loop(0, 14, six, 0)
    plsc.subcore_barrier()

    @pl.when(s < NS - 1)
    def _():
        pltpu.sync_copy(acc.at[pl.ds(s * RPT, RPT)],
                        part.at[c, pl.ds(s * RPT, RPT)])

    @pl.when(s == NS - 1)
    def _():
        pltpu.sync_copy(acc.at[pl.ds(15 * RPT, RLAST)],
                        part.at[c, pl.ds(15 * RPT, RLAST)])


GB = 8
BR = GB * 128


def _diag_scale(dp, x):
    rr = lax.broadcasted_iota(jnp.int32, (128, 128), 0)
    cc = lax.broadcasted_iota(jnp.int32, (128, 128), 1)
    outs = []
    for g in range(GB):
        iv = 1.0 / (1.0 + dp[0, g, :] + dp[1, g, :])
        dg = jnp.where(rr == cc,
                       jnp.broadcast_to(iv[None, :], (128, 128)), 0.0)
        outs.append(jnp.dot(dg, x[g * 128:(g + 1) * 128, :],
                            preferred_element_type=jnp.float32))
    return jnp.concatenate(outs, axis=0)


def _combine_body(dp_ref, p_ref, x_ref, o_ref):
    o_ref[...] = (p_ref[0] + p_ref[1]
                  + _diag_scale(dp_ref[...], x_ref[...]))


_combine_call = pl.pallas_call(
    _combine_body,
    grid=(NP2 // BR,),
    in_specs=[
        pl.BlockSpec((NC, GB, 128), lambda j: (0, j, 0)),
        pl.BlockSpec((NC, BR, D), lambda j: (0, j, 0)),
        pl.BlockSpec((BR, D), lambda j: (j, 0)),
    ],
    out_specs=pl.BlockSpec((BR, D), lambda j: (j, 0)),
    out_shape=jax.ShapeDtypeStruct((NP2, D), jnp.float32),
)


def _final_body(dp_ref, q_ref, x1_ref, w_ref, b_ref, o_ref):
    h = q_ref[0] + q_ref[1] + _diag_scale(dp_ref[...], x1_ref[...])
    o_ref[...] = lax.dot_general(
        h, w_ref[...], (((1,), (1,)), ((), ())),
        preferred_element_type=jnp.float32) + b_ref[...]


_final_call = pl.pallas_call(
    _final_body,
    grid=(NP2 // BR,),
    in_specs=[
        pl.BlockSpec((NC, GB, 128), lambda j: (0, j, 0)),
        pl.BlockSpec((NC, BR, D), lambda j: (0, j, 0)),
        pl.BlockSpec((BR, D), lambda j: (j, 0)),
        pl.BlockSpec((CO, D), lambda j: (0, 0)),
        pl.BlockSpec((1, CO), lambda j: (0, 0)),
    ],
    out_specs=pl.BlockSpec((BR, CO), lambda j: (j, 0)),
    out_shape=jax.ShapeDtypeStruct((N, CO), jnp.float32),
)


def kernel(data, edge_index, edge_weight, W, b):
    row = edge_index[0]
    col = edge_index[1]
    npad = EPAD - E
    pidx = jnp.arange(npad, dtype=jnp.int32)
    rowp = jnp.concatenate([row, pidx]).reshape(NW, NCHUNK, B)
    colp = jnp.concatenate([col, pidx]).reshape(NW, NCHUNK, B)
    ewp = jnp.concatenate(
        [edge_weight, jnp.zeros((npad,), jnp.float32)]).reshape(NW, NCHUNK, B)
    xp = jnp.concatenate(
        [data, jnp.zeros((NP2 - N, D), jnp.float32)], axis=0)
    zpd = jnp.zeros((NPD,), jnp.float32)

    degpart = _deg_kernel(colp, ewp, zpd)
    wn = _wn_kernel(degpart, rowp, colp, ewp)
    dpr = degpart.reshape(NC, NPD // 128, 128)

    wni = lax.bitcast_convert_type(wn, jnp.int32)
    meta = jnp.stack([colp, rowp, wni], axis=2)

    part1 = _spmm_kernel(xp, meta)
    x1 = _combine_call(dpr, part1, xp)
    part2 = _spmm_kernel(x1, meta)
    return _final_call(dpr, part2, x1, W, b.reshape(1, CO))
